# Initial kernel scaffold; baseline (speedup 1.0000x reference)
import functools

import jax
import jax.numpy as jnp
from jax import lax
from jax.experimental import pallas as pl
from jax.experimental.pallas import tpu as pltpu
from jax.experimental.pallas import tpu_sc as plsc

N = 50000
F = 770
K = 4096
E = 1600000

NC = 2
NS = 16
NW = NC * NS

NT = 3136
NPAD = NT * NS
VR = NT // 16

DMAX = 1024
DVR = DMAX // 16

EW = 128
ER_W = 391
ER_TOT = ER_W * NW
EPAD = ER_TOT * EW
NH = NPAD + 16

KPW = K // NW

_mesh = plsc.VectorSubcoreMesh(core_axis_name="c", subcore_axis_name="s")


@functools.partial(
    pl.kernel,
    mesh=_mesh,
    out_type=jax.ShapeDtypeStruct((NC, NPAD), jnp.float32),
    scratch_types=[
        pltpu.VMEM((ER_W, EW), jnp.int32),
        pltpu.VMEM((EW,), jnp.float32),
        pltpu.VMEM((NT,), jnp.float32),
        pltpu.VMEM_SHARED((NH,), jnp.float32),
        pltpu.SemaphoreType.DMA,
    ],
)
def _bincount(t_hbm, part_hbm, rows_v, ones_v, zb_v, hist_sh, sem):
    c = lax.axis_index("c")
    s = lax.axis_index("s")
    w = c * NS + s

    ones16 = jnp.ones((16,), jnp.float32)
    zero16 = jnp.zeros((16,), jnp.float32)

    def _fill_ones(i, carry):
        ones_v[pl.ds(i * 16, 16)] = ones16
        return carry

    lax.fori_loop(0, EW // 16, _fill_ones, 0)

    def _fill_zeros(i, carry):
        zb_v[pl.ds(i * 16, 16)] = zero16
        return carry

    lax.fori_loop(0, NT // 16, _fill_zeros, 0)

    pltpu.sync_copy(zb_v, hist_sh.at[pl.ds(s * NT, NT)])

    @pl.when(s == 0)
    def _zero_tail():
        pltpu.sync_copy(zb_v.at[pl.ds(0, 16)], hist_sh.at[pl.ds(NPAD, 16)])

    plsc.subcore_barrier()

    pltpu.sync_copy(t_hbm.at[pl.ds(w * ER_W, ER_W)], rows_v)

    def _fire(j, carry):
        pltpu.async_copy(ones_v, hist_sh.at[rows_v.at[j]], sem, add=True)
        return carry

    lax.fori_loop(0, ER_W, _fire, 0)
    pltpu.make_async_copy(t_hbm.at[pl.ds(w * ER_W, ER_W)], rows_v, sem).wait()

    plsc.subcore_barrier()
    pltpu.sync_copy(hist_sh.at[pl.ds(s * NT, NT)], part_hbm.at[c, pl.ds(s * NT, NT)])


@functools.partial(
    pl.kernel,
    mesh=_mesh,
    out_type=jax.ShapeDtypeStruct((K, F), jnp.float32),
    scratch_types=[
        pltpu.VMEM((NT,), jnp.float32),
        pltpu.VMEM((NT,), jnp.float32),
        pltpu.VMEM((NT,), jnp.int32),
        pltpu.VMEM((DMAX,), jnp.int32),
        pltpu.VMEM((DMAX,), jnp.int32),
        pltpu.VMEM((DMAX,), jnp.int32),
        pltpu.VMEM((DMAX,), jnp.int32),
        pltpu.VMEM((K,), jnp.int32),
        pltpu.VMEM((256,), jnp.int32),
        pltpu.VMEM((256,), jnp.int32),
        pltpu.VMEM((KPW,), jnp.int32),
        pltpu.VMEM((KPW, F), jnp.float32),
        pltpu.VMEM_SHARED((NS, DMAX), jnp.int32),
        pltpu.VMEM_SHARED((NS, K), jnp.int32),
        pltpu.VMEM_SHARED((K,), jnp.int32),
        pltpu.SemaphoreType.DMA,
    ],
)
def _select_gather(part_hbm, x_hbm, out_hbm, p0_v, p1_v, bins_v, hist_v,
                   row_v, pre_v, tot_v, sel_v, mrg_v, mrw_v, idx_v, rows_v,
                   grid_sh, selgrid_sh, outsp_sh, sem):
    c = lax.axis_index("c")
    s = lax.axis_index("s")

    pltpu.sync_copy(part_hbm.at[0, pl.ds(s * NT, NT)], p0_v)
    pltpu.sync_copy(part_hbm.at[1, pl.ds(s * NT, NT)], p1_v)

    zero16 = jnp.zeros((16,), jnp.int32)

    def _zero_hist(i, carry):
        hist_v[pl.ds(i * 16, 16)] = zero16
        pre_v[pl.ds(i * 16, 16)] = zero16
        tot_v[pl.ds(i * 16, 16)] = zero16
        return carry

    lax.fori_loop(0, DVR, _zero_hist, 0)

    cap = jnp.float32(DMAX - 1)

    def _hist(k, carry):
        d = p0_v[pl.ds(k * 16, 16)] + p1_v[pl.ds(k * 16, 16)]
        b = jnp.minimum(d, cap).astype(jnp.int32)
        bins_v[pl.ds(k * 16, 16)] = b
        cnt, last = plsc.scan_count(b)
        plsc.addupdate_scatter(hist_v, [b], cnt, mask=last)
        return carry

    lax.fori_loop(0, VR, _hist, 0)

    pltpu.sync_copy(hist_v, grid_sh.at[s])
    plsc.subcore_barrier()

    def _acc(s2, carry):
        pltpu.sync_copy(grid_sh.at[s2], row_v)
        wsel = jnp.where(s2 < s, jnp.int32(1), jnp.int32(0))

        def _acc_inner(i, carry2):
            r = row_v[pl.ds(i * 16, 16)]
            tot_v[pl.ds(i * 16, 16)] = tot_v[pl.ds(i * 16, 16)] + r
            pre_v[pl.ds(i * 16, 16)] = pre_v[pl.ds(i * 16, 16)] + r * wsel
            return carry2

        return lax.fori_loop(0, DVR, _acc_inner, carry)

    lax.fori_loop(0, NS, _acc, 0)

    def _suffix(i, carry):
        q = DVR - 1 - i
        t = tot_v[pl.ds(q * 16, 16)]
        cs = plsc.cumsum(t)
        total = jnp.sum(t)
        start = (carry + total) - cs
        tot_v[pl.ds(q * 16, 16)] = start + pre_v[pl.ds(q * 16, 16)]
        return carry + total

    lax.fori_loop(0, DVR, _suffix, jnp.int32(0))

    def _zero_sel(i, carry):
        sel_v[pl.ds(i * 16, 16)] = zero16
        return carry

    lax.fori_loop(0, K // 16, _zero_sel, 0)

    iota16 = lax.iota(jnp.int32, 16)

    def _scatter(k, carry):
        b = bins_v[pl.ds(k * 16, 16)]
        cnt, last = plsc.scan_count(b)
        base = plsc.load_gather(tot_v, [b])
        pos = base + cnt - 1
        node = s * NT + k * 16 + iota16
        selmask = pos < K
        posw = jnp.where(selmask, pos, 0)
        plsc.store_scatter(sel_v, [posw], node, mask=selmask)
        plsc.addupdate_scatter(tot_v, [b], cnt, mask=last)
        return carry

    lax.fori_loop(0, VR, _scatter, 0)

    pltpu.sync_copy(sel_v, selgrid_sh.at[s])
    plsc.subcore_barrier()

    def _zero_mrg(i, carry):
        mrg_v[pl.ds(i * 16, 16)] = zero16
        return carry

    lax.fori_loop(0, 256 // 16, _zero_mrg, 0)

    def _merge(s2, carry):
        pltpu.sync_copy(selgrid_sh.at[s2, pl.ds(s * 256, 256)], mrw_v)

        def _merge_inner(i, carry2):
            mrg_v[pl.ds(i * 16, 16)] = mrg_v[pl.ds(i * 16, 16)] + mrw_v[pl.ds(i * 16, 16)]
            return carry2

        return lax.fori_loop(0, 256 // 16, _merge_inner, carry)

    lax.fori_loop(0, NS, _merge, 0)

    pltpu.sync_copy(mrg_v, outsp_sh.at[pl.ds(s * 256, 256)])
    plsc.subcore_barrier()

    g0 = c * (NS * KPW) + s * KPW
    pltpu.sync_copy(outsp_sh.at[pl.ds(g0, KPW)], idx_v)
    pltpu.async_copy(x_hbm.at[idx_v], rows_v, sem).wait()
    pltpu.sync_copy(rows_v, out_hbm.at[pl.ds(g0, KPW)])


def kernel(x, edge_index):
    t = edge_index[1].astype(jnp.int32)
    t = jnp.concatenate([t, jnp.full((EPAD - E,), NPAD, jnp.int32)])
    part = _bincount(t.reshape(ER_TOT, EW))
    return _select_gather(part, x)


# speedup vs baseline: 8.1102x; 8.1102x over previous
import functools

import jax
import jax.numpy as jnp
from jax import lax
from jax.experimental import pallas as pl
from jax.experimental.pallas import tpu as pltpu
from jax.experimental.pallas import tpu_sc as plsc

N = 50000
F = 770
K = 4096
E = 1600000

NC = 2
NS = 16
NW = NC * NS

NT = 3136
NPAD = NT * NS
VR = NT // 16

DMAX = 1024
DVR = DMAX // 16

EW = 128
ER_W = 392
ER_TOT = ER_W * NW
EPAD = ER_TOT * EW
NH = NPAD + 16

KPW = K // NW

_mesh = plsc.VectorSubcoreMesh(core_axis_name="c", subcore_axis_name="s")
_params = pltpu.CompilerParams(needs_layout_passes=False)


@functools.partial(
    pl.kernel,
    mesh=_mesh,
    out_type=jax.ShapeDtypeStruct((NC * NPAD,), jnp.float32),
    compiler_params=_params,
    scratch_types=[
        pltpu.VMEM((ER_W, EW), jnp.int32),
        pltpu.VMEM((EW,), jnp.float32),
        pltpu.VMEM((NT,), jnp.float32),
        pltpu.VMEM_SHARED((NH,), jnp.float32),
        pltpu.SemaphoreType.DMA,
    ],
)
def _bincount(t_hbm, part_hbm, rows_v, ones_v, zb_v, hist_sh, sem):
    c = lax.axis_index("c")
    s = lax.axis_index("s")
    w = c * NS + s

    ones16 = jnp.ones((16,), jnp.float32)
    zero16 = jnp.zeros((16,), jnp.float32)

    def _fill_ones(i, carry):
        ones_v[pl.ds(i * 16, 16)] = ones16
        return carry

    lax.fori_loop(0, EW // 16, _fill_ones, 0)

    def _fill_zeros(i, carry):
        zb_v[pl.ds(i * 16, 16)] = zero16
        return carry

    lax.fori_loop(0, NT // 16, _fill_zeros, 0)

    pltpu.sync_copy(zb_v, hist_sh.at[pl.ds(s * NT, NT)])

    @pl.when(s == 0)
    def _zero_tail():
        pltpu.sync_copy(zb_v.at[pl.ds(0, 16)], hist_sh.at[pl.ds(NPAD, 16)])

    plsc.subcore_barrier()

    pltpu.sync_copy(t_hbm.at[pl.ds(w * ER_W, ER_W)], rows_v)

    def _fire(j, carry):
        pltpu.async_copy(ones_v, hist_sh.at[rows_v.at[j]], sem, add=True)
        return carry

    lax.fori_loop(0, ER_W, _fire, 0)
    pltpu.make_async_copy(t_hbm.at[pl.ds(w * ER_W, ER_W)], rows_v, sem).wait()

    plsc.subcore_barrier()
    pltpu.sync_copy(hist_sh.at[pl.ds(s * NT, NT)], zb_v)
    pltpu.sync_copy(zb_v, part_hbm.at[pl.ds(c * NPAD + s * NT, NT)])


FA = 640
FB = 128
FT = 128
FOUT = 896


@functools.partial(
    pl.kernel,
    mesh=_mesh,
    out_type=jax.ShapeDtypeStruct((K, FOUT), jnp.float32),
    compiler_params=_params,
    scratch_types=[
        pltpu.VMEM((NT,), jnp.float32),
        pltpu.VMEM((NT,), jnp.float32),
        pltpu.VMEM((DMAX,), jnp.int32),
        pltpu.VMEM((DMAX,), jnp.int32),
        pltpu.VMEM((DMAX,), jnp.int32),
        pltpu.VMEM((DMAX,), jnp.int32),
        pltpu.VMEM((K,), jnp.int32),
        pltpu.VMEM((256,), jnp.int32),
        pltpu.VMEM((256,), jnp.int32),
        pltpu.VMEM((KPW,), jnp.int32),
        pltpu.VMEM((KPW, FA), jnp.float32),
        pltpu.VMEM((KPW, FB), jnp.float32),
        pltpu.VMEM_SHARED((NS, DMAX), jnp.int32),
        pltpu.VMEM_SHARED((NS, K), jnp.int32),
        pltpu.VMEM_SHARED((K,), jnp.int32),
        pltpu.SemaphoreType.DMA,
    ],
)
def _select_gather(part_hbm, x_hbm, xt_hbm, out_hbm, p0_v, p1_v, hist_v,
                   row_v, pre_v, tot_v, sel_v, mrg_v, mrw_v, idx_v,
                   rowsa_v, rowsb_v,
                   grid_sh, selgrid_sh, outsp_sh, sem):
    c = lax.axis_index("c")
    s = lax.axis_index("s")

    pltpu.sync_copy(part_hbm.at[pl.ds(s * NT, NT)], p0_v)
    pltpu.sync_copy(part_hbm.at[pl.ds(NPAD + s * NT, NT)], p1_v)

    zero16 = jnp.zeros((16,), jnp.int32)

    def _zero_hist(i, carry):
        hist_v[pl.ds(i * 16, 16)] = zero16
        pre_v[pl.ds(i * 16, 16)] = zero16
        tot_v[pl.ds(i * 16, 16)] = zero16
        return carry

    lax.fori_loop(0, DVR, _zero_hist, 0)

    cap = jnp.float32(DMAX - 1)

    def _hist(k, carry):
        d = p0_v[pl.ds(k * 16, 16)] + p1_v[pl.ds(k * 16, 16)]
        b = jnp.minimum(d, cap).astype(jnp.int32)
        cnt, last = plsc.scan_count(b)
        plsc.addupdate_scatter(hist_v, [b], cnt, mask=last)
        return carry

    lax.fori_loop(0, VR, _hist, 0)

    pltpu.sync_copy(hist_v, grid_sh.at[s])
    plsc.subcore_barrier()

    def _acc(s2, carry):
        pltpu.sync_copy(grid_sh.at[s2], row_v)
        wsel = jnp.where(s2 < s, jnp.int32(1), jnp.int32(0))

        def _acc_inner(i, carry2):
            r = row_v[pl.ds(i * 16, 16)]
            tot_v[pl.ds(i * 16, 16)] = tot_v[pl.ds(i * 16, 16)] + r
            pre_v[pl.ds(i * 16, 16)] = pre_v[pl.ds(i * 16, 16)] + r * wsel
            return carry2

        return lax.fori_loop(0, DVR, _acc_inner, carry)

    lax.fori_loop(0, NS, _acc, 0)

    def _suffix(i, carry):
        q = DVR - 1 - i
        t = tot_v[pl.ds(q * 16, 16)]
        cs = plsc.cumsum(t)
        total = jnp.sum(t)
        start = (carry + total) - cs
        tot_v[pl.ds(q * 16, 16)] = start + pre_v[pl.ds(q * 16, 16)]
        return carry + total

    lax.fori_loop(0, DVR, _suffix, jnp.int32(0))

    def _zero_sel(i, carry):
        sel_v[pl.ds(i * 16, 16)] = zero16
        return carry

    lax.fori_loop(0, K // 16, _zero_sel, 0)

    iota16 = lax.iota(jnp.int32, 16)

    def _scatter(k, carry):
        d = p0_v[pl.ds(k * 16, 16)] + p1_v[pl.ds(k * 16, 16)]
        b = jnp.minimum(d, cap).astype(jnp.int32)
        cnt, last = plsc.scan_count(b)
        base = plsc.load_gather(tot_v, [b])
        pos = base + cnt - 1
        node = s * NT + k * 16 + iota16
        selmask = pos < K
        posw = jnp.where(selmask, pos, 0)
        plsc.store_scatter(sel_v, [posw], node, mask=selmask)
        plsc.addupdate_scatter(tot_v, [b], cnt, mask=last)
        return carry

    lax.fori_loop(0, VR, _scatter, 0)

    pltpu.sync_copy(sel_v, selgrid_sh.at[s])
    plsc.subcore_barrier()

    def _zero_mrg(i, carry):
        mrg_v[pl.ds(i * 16, 16)] = zero16
        return carry

    lax.fori_loop(0, 256 // 16, _zero_mrg, 0)

    def _merge(s2, carry):
        pltpu.sync_copy(selgrid_sh.at[s2, pl.ds(s * 256, 256)], mrw_v)

        def _merge_inner(i, carry2):
            mrg_v[pl.ds(i * 16, 16)] = mrg_v[pl.ds(i * 16, 16)] + mrw_v[pl.ds(i * 16, 16)]
            return carry2

        return lax.fori_loop(0, 256 // 16, _merge_inner, carry)

    lax.fori_loop(0, NS, _merge, 0)

    pltpu.sync_copy(mrg_v, outsp_sh.at[pl.ds(s * 256, 256)])
    plsc.subcore_barrier()

    g0 = c * (NS * KPW) + s * KPW
    pltpu.sync_copy(outsp_sh.at[pl.ds(g0, KPW)], idx_v)
    da = pltpu.async_copy(x_hbm.at[:, pl.ds(0, FA)].at[idx_v], rowsa_v, sem)
    db = pltpu.async_copy(x_hbm.at[:, pl.ds(FA, FB)].at[idx_v], rowsb_v, sem)
    da.wait()
    db.wait()
    pltpu.sync_copy(rowsa_v, out_hbm.at[pl.ds(g0, KPW), pl.ds(0, FA)])
    pltpu.sync_copy(rowsb_v, out_hbm.at[pl.ds(g0, KPW), pl.ds(FA, FB)])
    pltpu.async_copy(xt_hbm.at[idx_v], rowsb_v, sem).wait()
    pltpu.sync_copy(rowsb_v, out_hbm.at[pl.ds(g0, KPW), pl.ds(FA + FB, FT)])


def kernel(x, edge_index):
    t = edge_index[1].astype(jnp.int32)
    t = jnp.concatenate([t, jnp.full((EPAD - E,), NPAD, jnp.int32)])
    part = _bincount(t.reshape(ER_TOT, EW))
    xt = jnp.pad(x[:, FA + FB:], ((0, 0), (0, FT - (F - FA - FB))))
    out = _select_gather(part, x, xt)
    return out[:, :F]
